# baseline (speedup 1.0000x reference)
import jax
import jax.numpy as jnp
import numpy as np
from jax import lax
from jax.experimental import pallas as pl
from jax.experimental.pallas import tpu as pltpu
from jax.experimental.pallas import tpu_sc as plsc

B, N, G, S, TOKEN_DIM = 8, 32768, 256, 32, 256
NW = 32
RPW = (B * G) // NW
WPB = NW // B
M = B * G * S
EPS = 1e-5
BLKG = 64
BLKP = BLKG * S
GRID = (B * G) // BLKG
BIG = np.float32(1e30)


def _fps_body(px_ref, py_ref, pz_ref, fx_ref, fy_ref, fz_ref, dist_ref):
    px = px_ref[...]
    py = py_ref[...]
    pz = pz_ref[...]
    lane = lax.broadcasted_iota(jnp.int32, (B, N), 1)
    colg = lax.broadcasted_iota(jnp.int32, (B, G), 1)
    dist_ref[...] = jnp.full((B, N), 1e10, jnp.float32)
    fx_ref[...] = jnp.zeros((B, G), jnp.float32)
    fy_ref[...] = jnp.zeros((B, G), jnp.float32)
    fz_ref[...] = jnp.zeros((B, G), jnp.float32)

    def step(i, c):
        cx, cy, cz = c
        sel = colg == i
        fx_ref[...] = jnp.where(sel, cx, fx_ref[...])
        fy_ref[...] = jnp.where(sel, cy, fy_ref[...])
        fz_ref[...] = jnp.where(sel, cz, fz_ref[...])
        dxx = px - cx
        dyy = py - cy
        dzz = pz - cz
        d = (dxx * dxx + dyy * dyy) + dzz * dzz
        dist = jnp.minimum(dist_ref[...], d)
        dist_ref[...] = dist
        mx = jnp.max(dist, axis=1, keepdims=True)
        idx = jnp.min(jnp.where(dist == mx, lane, N), axis=1, keepdims=True)
        hit = lane == idx
        nx = jnp.sum(jnp.where(hit, px, 0.0), axis=1, keepdims=True)
        ny = jnp.sum(jnp.where(hit, py, 0.0), axis=1, keepdims=True)
        nz = jnp.sum(jnp.where(hit, pz, 0.0), axis=1, keepdims=True)
        return (nx, ny, nz)

    lax.fori_loop(0, G, step, (px[:, 0:1], py[:, 0:1], pz[:, 0:1]))


def _fps(px, py, pz):
    return pl.pallas_call(
        _fps_body,
        out_shape=[jax.ShapeDtypeStruct((B, G), jnp.float32)] * 3,
        scratch_shapes=[pltpu.VMEM((B, N), jnp.float32)],
    )(px, py, pz)


UNROLL = 4
RB = 8
CAP = 512


def _knn_sc_body(px_hbm, py_hbm, pz_hbm, cx_hbm, cy_hbm, cz_hbm, out_hbm,
                 px_v, py_v, pz_v, cx_v, cy_v, cz_v, stage_v, cand_v, candi_v):
    w = lax.axis_index("c") * 16 + lax.axis_index("s")
    b = w // WPB
    g0 = (w % WPB) * RPW
    pltpu.sync_copy(px_hbm.at[b], px_v)
    pltpu.sync_copy(py_hbm.at[b], py_v)
    pltpu.sync_copy(pz_hbm.at[b], pz_v)
    pltpu.sync_copy(cx_hbm.at[b, pl.ds(g0, RPW)], cx_v)
    pltpu.sync_copy(cy_hbm.at[b, pl.ds(g0, RPW)], cy_v)
    pltpu.sync_copy(cz_hbm.at[b, pl.ds(g0, RPW)], cz_v)
    lidx = lax.iota(jnp.int32, 16)

    def bfly_max(x):
        for s in (8, 4, 2, 1):
            x = jnp.maximum(x, x[lidx ^ s])
        return x

    def rowgrp(rg, _):
        ctr = []
        for ri in range(RB):
            rsplat = jnp.full((16,), rg * RB + ri, jnp.int32)
            ctr.append((plsc.load_gather(cx_v, [rsplat]),
                        plsc.load_gather(cy_v, [rsplat]),
                        plsc.load_gather(cz_v, [rsplat])))

        def dists(base, k):
            xs = px_v[pl.ds(base + k * 16, 16)]
            ys = py_v[pl.ds(base + k * 16, 16)]
            zs = pz_v[pl.ds(base + k * 16, 16)]
            out = []
            for (cxr, cyr, czr) in ctr:
                dx = cxr - xs
                dy = cyr - ys
                dz = czr - zs
                out.append((dx * dx + dy * dy) + dz * dz)
            return out

        def p1(j, st):
            st = list(st)
            ds = dists(j * 16, 0)
            for ri in range(RB):
                m1, m2 = st[2 * ri], st[2 * ri + 1]
                d = ds[ri]
                hi = jnp.maximum(m1, d)
                st[2 * ri] = jnp.minimum(m1, d)
                st[2 * ri + 1] = jnp.minimum(m2, hi)
            return tuple(st)

        bigv = jnp.full((16,), BIG, jnp.float32)
        mm = plsc.parallel_loop(0, N // 16, 1, unroll=8,
                                carry=(bigv,) * (2 * RB))(p1)
        that = [bfly_max(jnp.maximum(mm[2 * ri], mm[2 * ri + 1]))
                for ri in range(RB)]

        def p2(j, cnts):
            ds = dists(j * 16, 0)
            gi = j * 16 + lidx
            new = []
            for ri in range(RB):
                mk = ds[ri] <= that[ri]
                prefix = plsc.cumsum(jnp.where(mk, 1, 0))
                pcv = plsc.all_reduce_population_count(mk)
                pos = (ri * CAP + jnp.minimum(cnts[ri], CAP - 16)
                       + prefix - 1)
                plsc.store_scatter(cand_v, [pos], ds[ri], mask=mk)
                plsc.store_scatter(candi_v, [pos], gi, mask=mk)
                new.append(cnts[ri] + pcv[0])
            return tuple(new)

        cnts = plsc.parallel_loop(0, N // 16, 1, unroll=8,
                                  carry=(jnp.int32(0),) * RB)(p2)

        for ri in range(RB):
            cnt = cnts[ri]
            cxr, cyr, czr = ctr[ri]

            def chunk(n, st, ri=ri, cnt=cnt):
                v0, v1, i0, i1, tv = st
                valid = (n * 16 + lidx) < cnt
                d = jnp.where(valid, cand_v[pl.ds(ri * CAP + n * 16, 16)], BIG)
                gidx = candi_v[pl.ds(ri * CAP + n * 16, 16)]
                mask = d < tv

                def cond(st2):
                    return plsc.all_reduce_population_count(st2[0])[0] > 0

                def ins(st2):
                    mk, v0, v1, i0, i1, tv = st2
                    ffsv = plsc.all_reduce_ffs(mk)
                    lsel = lidx == ffsv
                    dval = d[ffsv]
                    gi = gidx[ffsv]
                    in0 = bfly_max(v0) == tv
                    eq0 = jnp.logical_and(v0 == tv, in0)
                    f0 = plsc.all_reduce_ffs(eq0)
                    r0 = jnp.logical_and(lidx == f0, in0)
                    v0 = jnp.where(r0, dval, v0)
                    i0 = jnp.where(r0, gi, i0)
                    in1 = jnp.logical_not(in0)
                    eq1 = jnp.logical_and(v1 == tv, in1)
                    f1 = plsc.all_reduce_ffs(eq1)
                    r1 = jnp.logical_and(lidx == f1, in1)
                    v1 = jnp.where(r1, dval, v1)
                    i1 = jnp.where(r1, gi, i1)
                    tv = jnp.maximum(bfly_max(v0), bfly_max(v1))
                    mk = jnp.logical_and(jnp.logical_and(mk, d < tv),
                                         jnp.logical_not(lsel))
                    return (mk, v0, v1, i0, i1, tv)

                _, v0, v1, i0, i1, tv = lax.while_loop(cond, ins,
                                                       (mask, v0, v1, i0, i1, tv))
                return (v0, v1, i0, i1, tv)

            bigv16 = jnp.full((16,), BIG, jnp.float32)
            init = (bigv16, bigv16,
                    jnp.zeros((16,), jnp.int32),
                    jnp.zeros((16,), jnp.int32), bigv16)
            nch = (jnp.minimum(cnt, CAP) + 15) // 16
            v0, v1, i0, i1, t = lax.fori_loop(0, nch, chunk, init)
            r = rg * RB + ri
            stage_v[r, pl.ds(0, 16)] = plsc.load_gather(px_v, [i0]) - cxr
            stage_v[r, pl.ds(16, 16)] = plsc.load_gather(px_v, [i1]) - cxr
            stage_v[r, pl.ds(32, 16)] = plsc.load_gather(py_v, [i0]) - cyr
            stage_v[r, pl.ds(48, 16)] = plsc.load_gather(py_v, [i1]) - cyr
            stage_v[r, pl.ds(64, 16)] = plsc.load_gather(pz_v, [i0]) - czr
            stage_v[r, pl.ds(80, 16)] = plsc.load_gather(pz_v, [i1]) - czr
        return 0

    lax.fori_loop(0, RPW // RB, rowgrp, 0)
    pltpu.sync_copy(stage_v, out_hbm.at[w])


def _knn_sc(px, py, pz, cx, cy, cz):
    mesh = plsc.VectorSubcoreMesh(core_axis_name="c", subcore_axis_name="s")
    f = pl.kernel(
        _knn_sc_body,
        out_type=jax.ShapeDtypeStruct((NW, RPW, 96), jnp.float32),
        mesh=mesh,
        scratch_types=[
            pltpu.VMEM((N,), jnp.float32),
            pltpu.VMEM((N,), jnp.float32),
            pltpu.VMEM((N,), jnp.float32),
            pltpu.VMEM((RPW,), jnp.float32),
            pltpu.VMEM((RPW,), jnp.float32),
            pltpu.VMEM((RPW,), jnp.float32),
            pltpu.VMEM((RPW, 96), jnp.float32),
            pltpu.VMEM((RB * CAP,), jnp.float32),
            pltpu.VMEM((RB * CAP,), jnp.int32),
        ],
        compiler_params=pltpu.CompilerParams(needs_layout_passes=False),
    )
    return f(px, py, pz, cx, cy, cz)


def _h1(f3, w1t, b1):
    dx = f3[:, 0:1]
    dy = f3[:, 1:2]
    dz = f3[:, 2:3]
    nr = jnp.sqrt((dx * dx + dy * dy) + dz * dz)
    return (dx * w1t[0:1, :] + dy * w1t[1:2, :] + dz * w1t[2:3, :]
            + nr * w1t[3:4, :] + b1)


def _hstats_body(f3_ref, w1t_ref, b1_ref, out_ref):
    i = pl.program_id(0)
    h = _h1(f3_ref[...], w1t_ref[...], b1_ref[...])
    s1 = jnp.sum(h, axis=0, keepdims=True)
    s2 = jnp.sum(h * h, axis=0, keepdims=True)
    acc = jnp.concatenate([s1, s2, jnp.zeros((6, 128), jnp.float32)], axis=0)

    @pl.when(i == 0)
    def _():
        out_ref[...] = acc

    @pl.when(i > 0)
    def _():
        out_ref[...] = out_ref[...] + acc


def _hstats(f3, w1t, b1):
    return pl.pallas_call(
        _hstats_body,
        grid=(GRID,),
        in_specs=[
            pl.BlockSpec((BLKP, 3), lambda i: (i, 0)),
            pl.BlockSpec((4, 128), lambda i: (0, 0)),
            pl.BlockSpec((1, 128), lambda i: (0, 0)),
        ],
        out_specs=pl.BlockSpec((8, 128), lambda i: (0, 0)),
        out_shape=jax.ShapeDtypeStruct((8, 128), jnp.float32),
    )(f3, w1t, b1)


def _enc_main_body(f3_ref, st1_ref, w1t_ref, b1_ref, g1_ref, be1_ref,
                   w2t_ref, b2_ref, w3at_ref, w3bt_ref, b3_ref,
                   h3_ref, st2_ref):
    i = pl.program_id(0)
    st = st1_ref[...]
    mean1 = st[0:1, :] * (1.0 / M)
    var1 = st[1:2, :] * (1.0 / M) - mean1 * mean1
    sc1 = g1_ref[...] * lax.rsqrt(var1 + EPS)
    sh1 = be1_ref[...] - mean1 * sc1
    h = _h1(f3_ref[...], w1t_ref[...], b1_ref[...])
    h = jnp.maximum(h * sc1 + sh1, 0.0)
    h2 = jnp.dot(h.astype(jnp.bfloat16), w2t_ref[...],
                 preferred_element_type=jnp.float32) + b2_ref[...]
    fg = jnp.max(h2.reshape(BLKG, S, 256), axis=1)
    fgw = jnp.dot(fg.astype(jnp.bfloat16), w3at_ref[...],
                  preferred_element_type=jnp.float32)
    h3 = jnp.dot(h2.astype(jnp.bfloat16), w3bt_ref[...],
                 preferred_element_type=jnp.float32)
    h3 = (h3 + jnp.broadcast_to(fgw[:, None, :], (BLKG, S, 512)).reshape(BLKP, 512)
          + b3_ref[...])
    h3_ref[...] = h3.astype(jnp.bfloat16)
    s1 = jnp.sum(h3, axis=0, keepdims=True)
    s2 = jnp.sum(h3 * h3, axis=0, keepdims=True)
    acc = jnp.concatenate([s1, s2, jnp.zeros((6, 512), jnp.float32)], axis=0)

    @pl.when(i == 0)
    def _():
        st2_ref[...] = acc

    @pl.when(i > 0)
    def _():
        st2_ref[...] = st2_ref[...] + acc


def _enc_main(f3, st1, w1t, b1, g1, be1, w2t, b2, w3at, w3bt, b3):
    return pl.pallas_call(
        _enc_main_body,
        grid=(GRID,),
        in_specs=[
            pl.BlockSpec((BLKP, 3), lambda i: (i, 0)),
            pl.BlockSpec((8, 128), lambda i: (0, 0)),
            pl.BlockSpec((4, 128), lambda i: (0, 0)),
            pl.BlockSpec((1, 128), lambda i: (0, 0)),
            pl.BlockSpec((1, 128), lambda i: (0, 0)),
            pl.BlockSpec((1, 128), lambda i: (0, 0)),
            pl.BlockSpec((128, 256), lambda i: (0, 0)),
            pl.BlockSpec((1, 256), lambda i: (0, 0)),
            pl.BlockSpec((256, 512), lambda i: (0, 0)),
            pl.BlockSpec((256, 512), lambda i: (0, 0)),
            pl.BlockSpec((1, 512), lambda i: (0, 0)),
        ],
        out_specs=[
            pl.BlockSpec((BLKP, 512), lambda i: (i, 0)),
            pl.BlockSpec((8, 512), lambda i: (0, 0)),
        ],
        out_shape=[
            jax.ShapeDtypeStruct((M, 512), jnp.bfloat16),
            jax.ShapeDtypeStruct((8, 512), jnp.float32),
        ],
    )(f3, st1, w1t, b1, g1, be1, w2t, b2, w3at, w3bt, b3)


def _enc_final_body(h3_ref, st2_ref, g2_ref, be2_ref, w4t_ref, b4_ref, out_ref):
    st = st2_ref[...]
    mean2 = st[0:1, :] * (1.0 / M)
    var2 = st[1:2, :] * (1.0 / M) - mean2 * mean2
    sc2 = g2_ref[...] * lax.rsqrt(var2 + EPS)
    sh2 = be2_ref[...] - mean2 * sc2
    a = jnp.maximum(h3_ref[...].astype(jnp.float32) * sc2 + sh2, 0.0)
    tkn = jnp.dot(a.astype(jnp.bfloat16), w4t_ref[...],
                  preferred_element_type=jnp.float32) + b4_ref[...]
    out_ref[...] = jnp.max(tkn.reshape(BLKG, S, TOKEN_DIM), axis=1)


def _enc_final(h3, st2, g2, be2, w4t, b4):
    return pl.pallas_call(
        _enc_final_body,
        grid=(GRID,),
        in_specs=[
            pl.BlockSpec((BLKP, 512), lambda i: (i, 0)),
            pl.BlockSpec((8, 512), lambda i: (0, 0)),
            pl.BlockSpec((1, 512), lambda i: (0, 0)),
            pl.BlockSpec((1, 512), lambda i: (0, 0)),
            pl.BlockSpec((512, 256), lambda i: (0, 0)),
            pl.BlockSpec((1, 256), lambda i: (0, 0)),
        ],
        out_specs=pl.BlockSpec((BLKG, TOKEN_DIM), lambda i: (i, 0)),
        out_shape=jax.ShapeDtypeStruct((B * G, TOKEN_DIM), jnp.float32),
    )(h3, st2, g2, be2, w4t, b4)


def kernel(points, W1, b1, g1, be1, W2, b2, W3, b3, g2, be2, W4, b4):
    px = points[:, :, 0]
    py = points[:, :, 1]
    pz = points[:, :, 2]
    fx, fy, fz = _fps(px, py, pz)
    knn = _knn_sc(px, py, pz, fx, fy, fz)
    neigh = knn.reshape(NW, RPW, 3, 2, 16).transpose(0, 1, 3, 4, 2)
    f3 = neigh.reshape(M, 3)
    w1t = W1.T
    b1r = b1.reshape(1, 128)
    st1 = _hstats(f3, w1t, b1r)
    bf16 = jnp.bfloat16
    h3, st2 = _enc_main(f3, st1, w1t, b1r, g1.reshape(1, 128), be1.reshape(1, 128),
                        W2.T.astype(bf16), b2.reshape(1, 256),
                        W3[:, :256].T.astype(bf16), W3[:, 256:].T.astype(bf16),
                        b3.reshape(1, 512))
    tokens = _enc_final(h3, st2, g2.reshape(1, 512), be2.reshape(1, 512),
                        W4.T.astype(bf16), b4.reshape(1, 256))
    center = jnp.stack([fx, fy, fz], axis=-1)
    return tokens.reshape(B, G, TOKEN_DIM), center


# speedup vs baseline: 1.1791x; 1.1791x over previous
import jax
import jax.numpy as jnp
import numpy as np
from jax import lax
from jax.experimental import pallas as pl
from jax.experimental.pallas import tpu as pltpu
from jax.experimental.pallas import tpu_sc as plsc

B, N, G, S, TOKEN_DIM = 8, 32768, 256, 32, 256
NW = 32
RPW = (B * G) // NW
WPB = NW // B
M = B * G * S
EPS = 1e-5
BLKG = 64
BLKP = BLKG * S
GRID = (B * G) // BLKG
BIG = np.float32(1e30)


def _fps_body(px_ref, py_ref, pz_ref, fx_ref, fy_ref, fz_ref, dist_ref):
    px = px_ref[...]
    py = py_ref[...]
    pz = pz_ref[...]
    lane = lax.broadcasted_iota(jnp.int32, (B, N), 1)
    colg = lax.broadcasted_iota(jnp.int32, (B, G), 1)
    dist_ref[...] = jnp.full((B, N), 1e10, jnp.float32)
    fx_ref[...] = jnp.zeros((B, G), jnp.float32)
    fy_ref[...] = jnp.zeros((B, G), jnp.float32)
    fz_ref[...] = jnp.zeros((B, G), jnp.float32)

    def step(i, c):
        cx, cy, cz = c
        sel = colg == i
        fx_ref[...] = jnp.where(sel, cx, fx_ref[...])
        fy_ref[...] = jnp.where(sel, cy, fy_ref[...])
        fz_ref[...] = jnp.where(sel, cz, fz_ref[...])
        dxx = px - cx
        dyy = py - cy
        dzz = pz - cz
        d = (dxx * dxx + dyy * dyy) + dzz * dzz
        dist = jnp.minimum(dist_ref[...], d)
        dist_ref[...] = dist
        mx = jnp.max(dist, axis=1, keepdims=True)
        idx = jnp.min(jnp.where(dist == mx, lane, N), axis=1, keepdims=True)
        hit = lane == idx
        nx = jnp.sum(jnp.where(hit, px, 0.0), axis=1, keepdims=True)
        ny = jnp.sum(jnp.where(hit, py, 0.0), axis=1, keepdims=True)
        nz = jnp.sum(jnp.where(hit, pz, 0.0), axis=1, keepdims=True)
        return (nx, ny, nz)

    lax.fori_loop(0, G, step, (px[:, 0:1], py[:, 0:1], pz[:, 0:1]))


def _fps(px, py, pz):
    return pl.pallas_call(
        _fps_body,
        out_shape=[jax.ShapeDtypeStruct((B, G), jnp.float32)] * 3,
        scratch_shapes=[pltpu.VMEM((B, N), jnp.float32)],
    )(px, py, pz)


UNROLL = 4
RB = 8
CAP = 512


def _knn_sc_body(px_hbm, py_hbm, pz_hbm, cx_hbm, cy_hbm, cz_hbm, out_hbm,
                 px_v, py_v, pz_v, cx_v, cy_v, cz_v, stage_v, cand_v, candi_v):
    w = lax.axis_index("c") * 16 + lax.axis_index("s")
    b = w // WPB
    g0 = (w % WPB) * RPW
    pltpu.sync_copy(px_hbm.at[b], px_v)
    pltpu.sync_copy(py_hbm.at[b], py_v)
    pltpu.sync_copy(pz_hbm.at[b], pz_v)
    pltpu.sync_copy(cx_hbm.at[b, pl.ds(g0, RPW)], cx_v)
    pltpu.sync_copy(cy_hbm.at[b, pl.ds(g0, RPW)], cy_v)
    pltpu.sync_copy(cz_hbm.at[b, pl.ds(g0, RPW)], cz_v)
    lidx = lax.iota(jnp.int32, 16)

    def bfly_max(x):
        for s in (8, 4, 2, 1):
            x = jnp.maximum(x, x[lidx ^ s])
        return x

    def rowgrp(rg, _):
        ctr = []
        for ri in range(RB):
            rsplat = jnp.full((16,), rg * RB + ri, jnp.int32)
            ctr.append((plsc.load_gather(cx_v, [rsplat]),
                        plsc.load_gather(cy_v, [rsplat]),
                        plsc.load_gather(cz_v, [rsplat])))

        def dists(base, k):
            xs = px_v[pl.ds(base + k * 16, 16)]
            ys = py_v[pl.ds(base + k * 16, 16)]
            zs = pz_v[pl.ds(base + k * 16, 16)]
            out = []
            for (cxr, cyr, czr) in ctr:
                dx = cxr - xs
                dy = cyr - ys
                dz = czr - zs
                out.append((dx * dx + dy * dy) + dz * dz)
            return out

        def p1(j, st):
            st = list(st)
            ds = dists(j * 16, 0)
            for ri in range(RB):
                m1, m2 = st[2 * ri], st[2 * ri + 1]
                d = ds[ri]
                hi = jnp.maximum(m1, d)
                st[2 * ri] = jnp.minimum(m1, d)
                st[2 * ri + 1] = jnp.minimum(m2, hi)
            return tuple(st)

        bigv = jnp.full((16,), BIG, jnp.float32)
        mm = plsc.parallel_loop(0, N // 16, 1, unroll=8,
                                carry=(bigv,) * (2 * RB))(p1)
        that = [bfly_max(jnp.maximum(mm[2 * ri], mm[2 * ri + 1]))
                for ri in range(RB)]

        def p2(j, cnts):
            ds = dists(j * 16, 0)
            gi = j * 16 + lidx
            new = []
            for ri in range(RB):
                mk = ds[ri] <= that[ri]
                prefix = plsc.cumsum(jnp.where(mk, 1, 0))
                pcv = plsc.all_reduce_population_count(mk)
                pos = (ri * CAP + jnp.minimum(cnts[ri], CAP - 16)
                       + prefix - 1)
                plsc.store_scatter(cand_v, [pos], ds[ri], mask=mk)
                plsc.store_scatter(candi_v, [pos], gi, mask=mk)
                new.append(cnts[ri] + pcv[0])
            return tuple(new)

        cnts = plsc.parallel_loop(0, N // 16, 1, unroll=4,
                                  carry=(jnp.int32(0),) * RB)(p2)

        for ri in range(RB):
            cnt = cnts[ri]
            cxr, cyr, czr = ctr[ri]

            def chunk(n, st, ri=ri, cnt=cnt):
                v0, v1, i0, i1, tv = st
                valid = (n * 16 + lidx) < cnt
                d = jnp.where(valid, cand_v[pl.ds(ri * CAP + n * 16, 16)], BIG)
                gidx = candi_v[pl.ds(ri * CAP + n * 16, 16)]
                mask = d < tv

                def cond(st2):
                    return plsc.all_reduce_population_count(st2[0])[0] > 0

                def ins(st2):
                    mk, v0, v1, i0, i1, tv = st2
                    ffsv = plsc.all_reduce_ffs(mk)
                    lsel = lidx == ffsv
                    dval = d[ffsv]
                    gi = gidx[ffsv]
                    in0 = bfly_max(v0) == tv
                    eq0 = jnp.logical_and(v0 == tv, in0)
                    f0 = plsc.all_reduce_ffs(eq0)
                    r0 = jnp.logical_and(lidx == f0, in0)
                    v0 = jnp.where(r0, dval, v0)
                    i0 = jnp.where(r0, gi, i0)
                    in1 = jnp.logical_not(in0)
                    eq1 = jnp.logical_and(v1 == tv, in1)
                    f1 = plsc.all_reduce_ffs(eq1)
                    r1 = jnp.logical_and(lidx == f1, in1)
                    v1 = jnp.where(r1, dval, v1)
                    i1 = jnp.where(r1, gi, i1)
                    tv = jnp.maximum(bfly_max(v0), bfly_max(v1))
                    mk = jnp.logical_and(jnp.logical_and(mk, d < tv),
                                         jnp.logical_not(lsel))
                    return (mk, v0, v1, i0, i1, tv)

                _, v0, v1, i0, i1, tv = lax.while_loop(cond, ins,
                                                       (mask, v0, v1, i0, i1, tv))
                return (v0, v1, i0, i1, tv)

            bigv16 = jnp.full((16,), BIG, jnp.float32)
            init = (bigv16, bigv16,
                    jnp.zeros((16,), jnp.int32),
                    jnp.zeros((16,), jnp.int32), bigv16)
            nch = (jnp.minimum(cnt, CAP) + 15) // 16
            v0, v1, i0, i1, t = lax.fori_loop(0, nch, chunk, init)
            r = rg * RB + ri
            stage_v[r, pl.ds(0, 16)] = plsc.load_gather(px_v, [i0]) - cxr
            stage_v[r, pl.ds(16, 16)] = plsc.load_gather(px_v, [i1]) - cxr
            stage_v[r, pl.ds(32, 16)] = plsc.load_gather(py_v, [i0]) - cyr
            stage_v[r, pl.ds(48, 16)] = plsc.load_gather(py_v, [i1]) - cyr
            stage_v[r, pl.ds(64, 16)] = plsc.load_gather(pz_v, [i0]) - czr
            stage_v[r, pl.ds(80, 16)] = plsc.load_gather(pz_v, [i1]) - czr
        return 0

    lax.fori_loop(0, RPW // RB, rowgrp, 0)
    pltpu.sync_copy(stage_v, out_hbm.at[w])


def _knn_sc(px, py, pz, cx, cy, cz):
    mesh = plsc.VectorSubcoreMesh(core_axis_name="c", subcore_axis_name="s")
    f = pl.kernel(
        _knn_sc_body,
        out_type=jax.ShapeDtypeStruct((NW, RPW, 96), jnp.float32),
        mesh=mesh,
        scratch_types=[
            pltpu.VMEM((N,), jnp.float32),
            pltpu.VMEM((N,), jnp.float32),
            pltpu.VMEM((N,), jnp.float32),
            pltpu.VMEM((RPW,), jnp.float32),
            pltpu.VMEM((RPW,), jnp.float32),
            pltpu.VMEM((RPW,), jnp.float32),
            pltpu.VMEM((RPW, 96), jnp.float32),
            pltpu.VMEM((RB * CAP,), jnp.float32),
            pltpu.VMEM((RB * CAP,), jnp.int32),
        ],
        compiler_params=pltpu.CompilerParams(needs_layout_passes=False),
    )
    return f(px, py, pz, cx, cy, cz)


def _h1(f3, w1t, b1):
    dx = f3[:, 0:1]
    dy = f3[:, 1:2]
    dz = f3[:, 2:3]
    nr = jnp.sqrt((dx * dx + dy * dy) + dz * dz)
    return (dx * w1t[0:1, :] + dy * w1t[1:2, :] + dz * w1t[2:3, :]
            + nr * w1t[3:4, :] + b1)


def _hstats_body(f3_ref, w1t_ref, b1_ref, out_ref):
    i = pl.program_id(0)
    h = _h1(f3_ref[...], w1t_ref[...], b1_ref[...])
    s1 = jnp.sum(h, axis=0, keepdims=True)
    s2 = jnp.sum(h * h, axis=0, keepdims=True)
    acc = jnp.concatenate([s1, s2, jnp.zeros((6, 128), jnp.float32)], axis=0)

    @pl.when(i == 0)
    def _():
        out_ref[...] = acc

    @pl.when(i > 0)
    def _():
        out_ref[...] = out_ref[...] + acc


def _hstats(f3, w1t, b1):
    return pl.pallas_call(
        _hstats_body,
        grid=(GRID,),
        in_specs=[
            pl.BlockSpec((BLKP, 3), lambda i: (i, 0)),
            pl.BlockSpec((4, 128), lambda i: (0, 0)),
            pl.BlockSpec((1, 128), lambda i: (0, 0)),
        ],
        out_specs=pl.BlockSpec((8, 128), lambda i: (0, 0)),
        out_shape=jax.ShapeDtypeStruct((8, 128), jnp.float32),
    )(f3, w1t, b1)


def _enc_main_body(f3_ref, st1_ref, w1t_ref, b1_ref, g1_ref, be1_ref,
                   w2t_ref, b2_ref, w3at_ref, w3bt_ref, b3_ref,
                   h3_ref, st2_ref):
    i = pl.program_id(0)
    st = st1_ref[...]
    mean1 = st[0:1, :] * (1.0 / M)
    var1 = st[1:2, :] * (1.0 / M) - mean1 * mean1
    sc1 = g1_ref[...] * lax.rsqrt(var1 + EPS)
    sh1 = be1_ref[...] - mean1 * sc1
    h = _h1(f3_ref[...], w1t_ref[...], b1_ref[...])
    h = jnp.maximum(h * sc1 + sh1, 0.0)
    h2 = jnp.dot(h.astype(jnp.bfloat16), w2t_ref[...],
                 preferred_element_type=jnp.float32) + b2_ref[...]
    fg = jnp.max(h2.reshape(BLKG, S, 256), axis=1)
    fgw = jnp.dot(fg.astype(jnp.bfloat16), w3at_ref[...],
                  preferred_element_type=jnp.float32)
    h3 = jnp.dot(h2.astype(jnp.bfloat16), w3bt_ref[...],
                 preferred_element_type=jnp.float32)
    h3 = (h3 + jnp.broadcast_to(fgw[:, None, :], (BLKG, S, 512)).reshape(BLKP, 512)
          + b3_ref[...])
    h3_ref[...] = h3.astype(jnp.bfloat16)
    s1 = jnp.sum(h3, axis=0, keepdims=True)
    s2 = jnp.sum(h3 * h3, axis=0, keepdims=True)
    acc = jnp.concatenate([s1, s2, jnp.zeros((6, 512), jnp.float32)], axis=0)

    @pl.when(i == 0)
    def _():
        st2_ref[...] = acc

    @pl.when(i > 0)
    def _():
        st2_ref[...] = st2_ref[...] + acc


def _enc_main(f3, st1, w1t, b1, g1, be1, w2t, b2, w3at, w3bt, b3):
    return pl.pallas_call(
        _enc_main_body,
        grid=(GRID,),
        in_specs=[
            pl.BlockSpec((BLKP, 3), lambda i: (i, 0)),
            pl.BlockSpec((8, 128), lambda i: (0, 0)),
            pl.BlockSpec((4, 128), lambda i: (0, 0)),
            pl.BlockSpec((1, 128), lambda i: (0, 0)),
            pl.BlockSpec((1, 128), lambda i: (0, 0)),
            pl.BlockSpec((1, 128), lambda i: (0, 0)),
            pl.BlockSpec((128, 256), lambda i: (0, 0)),
            pl.BlockSpec((1, 256), lambda i: (0, 0)),
            pl.BlockSpec((256, 512), lambda i: (0, 0)),
            pl.BlockSpec((256, 512), lambda i: (0, 0)),
            pl.BlockSpec((1, 512), lambda i: (0, 0)),
        ],
        out_specs=[
            pl.BlockSpec((BLKP, 512), lambda i: (i, 0)),
            pl.BlockSpec((8, 512), lambda i: (0, 0)),
        ],
        out_shape=[
            jax.ShapeDtypeStruct((M, 512), jnp.bfloat16),
            jax.ShapeDtypeStruct((8, 512), jnp.float32),
        ],
    )(f3, st1, w1t, b1, g1, be1, w2t, b2, w3at, w3bt, b3)


def _enc_final_body(h3_ref, st2_ref, g2_ref, be2_ref, w4t_ref, b4_ref, out_ref):
    st = st2_ref[...]
    mean2 = st[0:1, :] * (1.0 / M)
    var2 = st[1:2, :] * (1.0 / M) - mean2 * mean2
    sc2 = g2_ref[...] * lax.rsqrt(var2 + EPS)
    sh2 = be2_ref[...] - mean2 * sc2
    a = jnp.maximum(h3_ref[...].astype(jnp.float32) * sc2 + sh2, 0.0)
    tkn = jnp.dot(a.astype(jnp.bfloat16), w4t_ref[...],
                  preferred_element_type=jnp.float32) + b4_ref[...]
    out_ref[...] = jnp.max(tkn.reshape(BLKG, S, TOKEN_DIM), axis=1)


def _enc_final(h3, st2, g2, be2, w4t, b4):
    return pl.pallas_call(
        _enc_final_body,
        grid=(GRID,),
        in_specs=[
            pl.BlockSpec((BLKP, 512), lambda i: (i, 0)),
            pl.BlockSpec((8, 512), lambda i: (0, 0)),
            pl.BlockSpec((1, 512), lambda i: (0, 0)),
            pl.BlockSpec((1, 512), lambda i: (0, 0)),
            pl.BlockSpec((512, 256), lambda i: (0, 0)),
            pl.BlockSpec((1, 256), lambda i: (0, 0)),
        ],
        out_specs=pl.BlockSpec((BLKG, TOKEN_DIM), lambda i: (i, 0)),
        out_shape=jax.ShapeDtypeStruct((B * G, TOKEN_DIM), jnp.float32),
    )(h3, st2, g2, be2, w4t, b4)


def kernel(points, W1, b1, g1, be1, W2, b2, W3, b3, g2, be2, W4, b4):
    px = points[:, :, 0]
    py = points[:, :, 1]
    pz = points[:, :, 2]
    fx, fy, fz = _fps(px, py, pz)
    knn = _knn_sc(px, py, pz, fx, fy, fz)
    neigh = knn.reshape(NW, RPW, 3, 2, 16).transpose(0, 1, 3, 4, 2)
    f3 = neigh.reshape(M, 3)
    w1t = W1.T
    b1r = b1.reshape(1, 128)
    st1 = _hstats(f3, w1t, b1r)
    bf16 = jnp.bfloat16
    h3, st2 = _enc_main(f3, st1, w1t, b1r, g1.reshape(1, 128), be1.reshape(1, 128),
                        W2.T.astype(bf16), b2.reshape(1, 256),
                        W3[:, :256].T.astype(bf16), W3[:, 256:].T.astype(bf16),
                        b3.reshape(1, 512))
    tokens = _enc_final(h3, st2, g2.reshape(1, 512), be2.reshape(1, 512),
                        W4.T.astype(bf16), b4.reshape(1, 256))
    center = jnp.stack([fx, fy, fz], axis=-1)
    return tokens.reshape(B, G, TOKEN_DIM), center
